# baseline (speedup 1.0000x reference)
import functools

import jax
import jax.numpy as jnp
from jax import lax
from jax.experimental import pallas as pl
from jax.experimental.pallas import tpu as pltpu
from jax.experimental.pallas import tpu_sc as plsc


def _table_body(wt_ref, w1_ref, b1_ref, out_ref):
    e = jnp.maximum(wt_ref[...], 0.0)
    s = jnp.dot(w1_ref[...], e, preferred_element_type=jnp.float32)
    out_ref[...] = jnp.maximum(s[0] + b1_ref[0], 0.0)


def _build_table(w_emb_t, w1, b1):
    v = w_emb_t.shape[1]
    return pl.pallas_call(
        _table_body,
        out_shape=jax.ShapeDtypeStruct((v,), jnp.float32),
        in_specs=[
            pl.BlockSpec(memory_space=pltpu.VMEM),
            pl.BlockSpec(memory_space=pltpu.VMEM),
            pl.BlockSpec(memory_space=pltpu.SMEM),
        ],
        out_specs=pl.BlockSpec(memory_space=pltpu.VMEM),
    )(w_emb_t, w1, b1)


_SC_INFO = plsc.get_sparse_core_info()
_NC = _SC_INFO.num_cores
_NS = _SC_INFO.num_subcores
_NW = _NC * _NS
_L = _SC_INFO.num_lanes


@functools.cache
def _make_gather(lseq: int, b: int, v: int):
    assert b % (_NW * _L) == 0
    cols_per_w = b // _NW
    mesh = plsc.VectorSubcoreMesh(core_axis_name="c", subcore_axis_name="s")

    @functools.partial(
        pl.kernel,
        out_type=jax.ShapeDtypeStruct((lseq, 1, b), jnp.float32),
        mesh=mesh,
        compiler_params=pltpu.CompilerParams(needs_layout_passes=False),
        scratch_types=[
            pltpu.VMEM((lseq, cols_per_w), jnp.int32),
            pltpu.VMEM((lseq, cols_per_w), jnp.float32),
            pltpu.VMEM((v,), jnp.float32),
            pltpu.SemaphoreType.DMA,
            pltpu.SemaphoreType.DMA,
            pltpu.SemaphoreType.DMA,
        ],
    )
    def gather_kernel(
        table_hbm, xt_hbm, out_hbm, idx_v, out_v, table_v, sem0, sem1, sem_out
    ):
        half = cols_per_w // 2
        wid = lax.axis_index("s") * _NC + lax.axis_index("c")
        c0 = wid * cols_per_w
        a0 = pltpu.async_copy(
            xt_hbm.at[:, pl.ds(c0, half)], idx_v.at[:, pl.ds(0, half)], sem0
        )
        a1 = pltpu.async_copy(
            xt_hbm.at[:, pl.ds(c0 + half, half)],
            idx_v.at[:, pl.ds(half, half)],
            sem1,
        )
        pltpu.sync_copy(table_hbm, table_v)

        blk = 10 if lseq % 10 == 0 else 1

        def gather_cols(base_g):
            @plsc.parallel_loop(base_g, base_g + half // _L, unroll=2)
            def _gather_iter(g):
                off = pl.multiple_of(g.astype(jnp.int32) * _L, _L)
                for r0 in range(0, lseq, blk):
                    vals = [
                        plsc.load_gather(table_v, [idx_v[r, pl.ds(off, _L)]])
                        for r in range(r0, r0 + blk)
                    ]
                    for j, r in enumerate(range(r0, r0 + blk)):
                        out_v[r, pl.ds(off, _L)] = vals[j]

        a0.wait()
        gather_cols(0)
        b0 = pltpu.async_copy(
            out_v.at[:, pl.ds(0, half)],
            out_hbm.at[:, 0, pl.ds(c0, half)],
            sem_out,
        )
        a1.wait()
        gather_cols(half // _L)
        b1 = pltpu.async_copy(
            out_v.at[:, pl.ds(half, half)],
            out_hbm.at[:, 0, pl.ds(c0 + half, half)],
            sem_out,
        )
        b0.wait()
        b1.wait()

    return gather_kernel


def kernel(x, W_emb, W1, b1):
    B, Lseq = x.shape
    V, _ = W_emb.shape
    table = _build_table(
        W_emb.T.astype(jnp.float32), W1.astype(jnp.float32), b1.astype(jnp.float32)
    )
    out3 = _make_gather(Lseq, B, V)(table, x.T.astype(jnp.int32))
    return out3.transpose((2, 0, 1))


# speedup vs baseline: 1.0679x; 1.0679x over previous
import functools

import jax
import jax.numpy as jnp
from jax import lax
from jax.experimental import pallas as pl
from jax.experimental.pallas import tpu as pltpu
from jax.experimental.pallas import tpu_sc as plsc


def _table_body(wt_ref, w1_ref, b1_ref, out_ref):
    e = jnp.maximum(wt_ref[...], 0.0)
    s = jnp.dot(w1_ref[...], e, preferred_element_type=jnp.float32)
    out_ref[...] = jnp.maximum(s[0] + b1_ref[0], 0.0)


def _build_table(w_emb_t, w1, b1):
    v = w_emb_t.shape[1]
    return pl.pallas_call(
        _table_body,
        out_shape=jax.ShapeDtypeStruct((v,), jnp.float32),
        in_specs=[
            pl.BlockSpec(memory_space=pltpu.VMEM),
            pl.BlockSpec(memory_space=pltpu.VMEM),
            pl.BlockSpec(memory_space=pltpu.SMEM),
        ],
        out_specs=pl.BlockSpec(memory_space=pltpu.VMEM),
    )(w_emb_t, w1, b1)


_SC_INFO = plsc.get_sparse_core_info()
_NC = _SC_INFO.num_cores
_NS = _SC_INFO.num_subcores
_NW = _NC * _NS
_L = _SC_INFO.num_lanes


@functools.cache
def _make_gather(lseq: int, b: int, v: int):
    assert b % (_NW * _L) == 0
    cols_per_w = b // _NW
    mesh = plsc.VectorSubcoreMesh(core_axis_name="c", subcore_axis_name="s")

    @functools.partial(
        pl.kernel,
        out_type=jax.ShapeDtypeStruct((lseq, 1, b), jnp.float32),
        mesh=mesh,
        compiler_params=pltpu.CompilerParams(needs_layout_passes=False),
        scratch_types=[
            pltpu.VMEM((lseq, cols_per_w), jnp.int32),
            pltpu.VMEM((lseq, cols_per_w), jnp.float32),
            pltpu.VMEM((v,), jnp.float32),
            pltpu.SemaphoreType.DMA,
        ],
    )
    def gather_kernel(table_hbm, xt_hbm, out_hbm, idx_v, out_v, table_v, sem):
        wid = lax.axis_index("s") * _NC + lax.axis_index("c")
        c0 = wid * cols_per_w
        a_in = pltpu.async_copy(xt_hbm.at[:, pl.ds(c0, cols_per_w)], idx_v, sem)
        pltpu.sync_copy(table_hbm, table_v)
        a_in.wait()

        blk = 10 if lseq % 10 == 0 else 1

        @plsc.parallel_loop(0, cols_per_w // _L, unroll=2)
        def _gather_iter(g):
            off = pl.multiple_of(g.astype(jnp.int32) * _L, _L)
            for r0 in range(0, lseq, blk):
                vals = [
                    plsc.load_gather(table_v, [idx_v[r, pl.ds(off, _L)]])
                    for r in range(r0, r0 + blk)
                ]
                for j, r in enumerate(range(r0, r0 + blk)):
                    out_v[r, pl.ds(off, _L)] = vals[j]

        pltpu.sync_copy(out_v, out_hbm.at[:, 0, pl.ds(c0, cols_per_w)])

    return gather_kernel


def kernel(x, W_emb, W1, b1):
    B, Lseq = x.shape
    V, _ = W_emb.shape
    table = _build_table(
        W_emb.T.astype(jnp.float32), W1.astype(jnp.float32), b1.astype(jnp.float32)
    )
    out3 = _make_gather(Lseq, B, V)(table, x.T.astype(jnp.int32))
    return out3.transpose((2, 0, 1))
